# Initial kernel scaffold; baseline (speedup 1.0000x reference)
import functools

import jax
import jax.numpy as jnp
from jax import lax
from jax.experimental import pallas as pl
from jax.experimental.pallas import tpu as pltpu
from jax.experimental.pallas import tpu_sc as plsc

N = 100000
E = 1600000
B = 128
P = 112
NW = 32
M_PAD = 1703936
E_PAD = 1601536
NP_PAD = 102400
GP_N = 102400
GP_B = 1024

_MESH = dict(core_axis_name="c", subcore_axis_name="s")


def _mm_body(x_ref, w_ref, b_ref, o_ref):
    o_ref[...] = jnp.dot(x_ref[...], w_ref[...],
                         preferred_element_type=jnp.float32) + b_ref[0:1, :]


def tc_matmul(x, w, b, mblk):
    m, k = x.shape
    p = w.shape[1]
    assert m % mblk == 0
    return pl.pallas_call(
        _mm_body,
        grid=(m // mblk,),
        in_specs=[
            pl.BlockSpec((mblk, k), lambda i: (i, 0)),
            pl.BlockSpec((k, p), lambda i: (0, 0)),
            pl.BlockSpec((8, p), lambda i: (0, 0)),
        ],
        out_specs=pl.BlockSpec((mblk, p), lambda i: (i, 0)),
        out_shape=jax.ShapeDtypeStruct((m, p), jnp.float32),
    )(x, w, b)


def _loopattr_body(pa_ref, o_ref):
    q = pa_ref[0] + pa_ref[1]
    deg = jnp.maximum(q[:, 47:48], 1.0)
    o_ref[...] = q[:, :32] / deg


def tc_loopattr(p_attr):
    mblk = 800
    return pl.pallas_call(
        _loopattr_body,
        grid=(N // mblk,),
        in_specs=[pl.BlockSpec((2, mblk, 48), lambda i: (0, i, 0))],
        out_specs=pl.BlockSpec((mblk, 32), lambda i: (i, 0)),
        out_shape=jax.ShapeDtypeStruct((N, 32), jnp.float32),
    )(p_attr)


def _alpha_body(ms_ref, att_ref, o_ref):
    m = ms_ref[...]
    m = jnp.where(m >= 0, m, 0.2 * m)
    o_ref[...] = jnp.sum(m * att_ref[0:1, :], axis=1)


def tc_alpha(msum, attp, mblk):
    m = msum.shape[0]
    return pl.pallas_call(
        _alpha_body,
        grid=(m // mblk,),
        in_specs=[
            pl.BlockSpec((mblk, P), lambda i: (i, 0)),
            pl.BlockSpec((8, P), lambda i: (0, 0)),
        ],
        out_specs=pl.BlockSpec((mblk,), lambda i: (i,)),
        out_shape=jax.ShapeDtypeStruct((m,), jnp.float32),
    )(msum, attp)


def _maxred_body(a_ref, o_ref):
    o_ref[...] = jnp.full((8,), jnp.max(a_ref[...]), jnp.float32)


def tc_maxred(alpha):
    m = alpha.shape[0]
    return pl.pallas_call(
        _maxred_body,
        in_specs=[pl.BlockSpec((m,), lambda: (0,))],
        out_specs=pl.BlockSpec((8,), lambda: (0,)),
        out_shape=jax.ShapeDtypeStruct((8,), jnp.float32),
    )(alpha)


def _u_body(a_ref, c_ref, xls_ref, o_ref, *, mblk, m_real):
    i = pl.program_id(0)
    rows = i * mblk + lax.broadcasted_iota(jnp.int32, (mblk,), 0)
    w = jnp.exp(a_ref[...] - c_ref[0])
    w = jnp.where(rows < m_real, w, 0.0)
    o_ref[...] = w[:, None] * xls_ref[...]


def tc_u(alpha, cmax, xls, mblk, m_real):
    m = alpha.shape[0]
    return pl.pallas_call(
        functools.partial(_u_body, mblk=mblk, m_real=m_real),
        grid=(m // mblk,),
        in_specs=[
            pl.BlockSpec((mblk,), lambda i: (i,)),
            pl.BlockSpec((8,), lambda i: (0,)),
            pl.BlockSpec((mblk, P), lambda i: (i, 0)),
        ],
        out_specs=pl.BlockSpec((mblk, P), lambda i: (i, 0)),
        out_shape=jax.ShapeDtypeStruct((m, P), jnp.float32),
    )(alpha, cmax, xls)


def _h_body(pn_ref, b_ref, o_ref, *, relu):
    q = pn_ref[0] + pn_ref[1]
    inv = 1.0 / jnp.maximum(q[:, 111:112], 1e-16)
    h = q * inv + b_ref[0:1, :]
    if relu:
        h = jnp.maximum(h, 0.0)
    o_ref[...] = h


def tc_h(p_num, biasp, relu):
    mblk = 800
    return pl.pallas_call(
        functools.partial(_h_body, relu=relu),
        grid=(N // mblk,),
        in_specs=[
            pl.BlockSpec((2, mblk, P), lambda i: (0, i, 0)),
            pl.BlockSpec((8, P), lambda i: (0, 0)),
        ],
        out_specs=pl.BlockSpec((mblk, P), lambda i: (i, 0)),
        out_shape=jax.ShapeDtypeStruct((N, P), jnp.float32),
    )(p_num, biasp)


def _head_body(pp_ref, w_ref, b_ref, o_ref):
    q = pp_ref[0, :B, :] + pp_ref[1, :B, :]
    cnt = jnp.maximum(q[:, 111:112], 1.0)
    pooled = q / cnt
    out = jnp.dot(pooled, w_ref[...], preferred_element_type=jnp.float32)
    o_ref[...] = jax.nn.sigmoid(out + b_ref[0:1, :])


def tc_head(p_pool, wlinp, blinp):
    return pl.pallas_call(
        _head_body,
        in_specs=[
            pl.BlockSpec((2, GP_B, P), lambda: (0, 0, 0)),
            pl.BlockSpec((P, 8), lambda: (0, 0)),
            pl.BlockSpec((8, 8), lambda: (0, 0)),
        ],
        out_specs=pl.BlockSpec((B, 8), lambda: (0, 0)),
        out_shape=jax.ShapeDtypeStruct((B, 8), jnp.float32),
    )(p_pool, wlinp, blinp)


def _gather_body(xl_hbm, xr_hbm, ze_hbm, s_hbm, d_hbm, xls_hbm, msum_hbm,
                 sbuf, dbuf, xlb, xrb, zeb, sem, *, m_rows):
    wid = lax.axis_index("s") * 2 + lax.axis_index("c")
    pw = m_rows // NW
    sup = 256
    nsup = pw // sup
    base0 = wid * pw

    def step(t, _):
        base = base0 + t * sup
        g = base // 128
        pltpu.sync_copy(s_hbm.at[pl.ds(g, 2), :], sbuf)
        pltpu.sync_copy(d_hbm.at[pl.ds(g, 2), :], dbuf)
        cps = [
            pltpu.async_copy(xl_hbm.at[sbuf.at[0]], xlb.at[pl.ds(0, 128)], sem),
            pltpu.async_copy(xl_hbm.at[sbuf.at[1]], xlb.at[pl.ds(128, 128)], sem),
            pltpu.async_copy(xr_hbm.at[dbuf.at[0]], xrb.at[pl.ds(0, 128)], sem),
            pltpu.async_copy(xr_hbm.at[dbuf.at[1]], xrb.at[pl.ds(128, 128)], sem),
            pltpu.async_copy(ze_hbm.at[pl.ds(base, sup)], zeb, sem),
        ]
        for cp in cps:
            cp.wait()
        pltpu.sync_copy(xlb, xls_hbm.at[pl.ds(base, sup)])

        def row(r, _):
            for c in range(7):
                sl = pl.ds(c * 16, 16)
                xlb[r, sl] = xlb[r, sl] + xrb[r, sl] + zeb[r, sl]
            return 0

        lax.fori_loop(0, sup, row, 0)
        pltpu.sync_copy(xlb, msum_hbm.at[pl.ds(base, sup)])
        return 0

    lax.fori_loop(0, nsup, step, 0)


def sc_gather_add(xl, xr, ze, s2d, d2d, m_rows):
    kfn = pl.kernel(
        functools.partial(_gather_body, m_rows=m_rows),
        mesh=plsc.VectorSubcoreMesh(**_MESH),
        out_type=[
            jax.ShapeDtypeStruct((m_rows, P), jnp.float32),
            jax.ShapeDtypeStruct((m_rows, P), jnp.float32),
        ],
        scratch_types=[
            pltpu.VMEM((2, 128), jnp.int32),
            pltpu.VMEM((2, 128), jnp.int32),
            pltpu.VMEM((256, P), jnp.float32),
            pltpu.VMEM((256, P), jnp.float32),
            pltpu.VMEM((256, P), jnp.float32),
            pltpu.SemaphoreType.DMA,
        ],
    )
    return kfn(xl, xr, ze, s2d, d2d)


def _scatter_body(d_hbm, u_hbm, out_hbm, dbuf, ibuf, ubuf, zbuf, acc, sem,
                  *, m_rows, gp, nslc):
    cid = lax.axis_index("c")
    sid = lax.axis_index("s")
    wid = sid * 2 + cid
    pw = m_rows // NW
    ngrp = pw // 128
    nsup = ngrp // 8
    rem = ngrp % 8
    zrows = gp // 16

    iota16 = lax.iota(jnp.int32, 16)

    def fill_zbuf(r, _):
        zbuf[r, :] = jnp.zeros((16,), jnp.float32)
        return 0

    lax.fori_loop(0, 800, fill_zbuf, 0)

    def do_groups(f, sbase, cnt):
        for j in range(cnt):
            gab = (wid * ngrp) * 128 + (sbase + j) * 128
            pltpu.sync_copy(d_hbm.at[pl.ds((wid * ngrp + sbase + j), 1), :],
                            dbuf.at[pl.ds(j, 1), :])

            def mkidx(q, _, gab=gab, j=j):
                ibuf[j, pl.ds(q * 16, 16)] = (gab + q * 16 + iota16) * nslc + f
                return 0

            lax.fori_loop(0, 8, mkidx, 0)
        cps = []
        for j in range(cnt):
            cps.append(pltpu.async_copy(u_hbm.at[ibuf.at[j]],
                                        ubuf.at[pl.ds(j * 128, 128)], sem))
        for cp in cps:
            cp.wait()
        for j in range(cnt):
            pltpu.sync_copy(ubuf.at[pl.ds(j * 128, 128)],
                            acc.at[dbuf.at[j]], add=True)

    def slice_pass(f, _):
        r0 = sid * zrows
        done = 0
        while done < zrows:
            cz = min(800, zrows - done)
            pltpu.sync_copy(zbuf.at[pl.ds(0, cz)], acc.at[pl.ds(r0 + done, cz)])
            done += cz
        plsc.subcore_barrier()

        def sup_step(t, _):
            def emit(j, t=t):
                return None
            for j in range(8):
                pass
            do_groups(f, t * 8, 8)
            return 0

        lax.fori_loop(0, nsup, sup_step, 0)
        if rem:
            do_groups(f, nsup * 8, rem)
        plsc.subcore_barrier()
        done = 0
        while done < zrows:
            cz = min(800, zrows - done)
            pltpu.sync_copy(
                acc.at[pl.ds(r0 + done, cz)],
                out_hbm.at[cid, pl.ds(r0 + done, cz), pl.ds(f * 16, 16)])
            done += cz
        plsc.subcore_barrier()
        return 0

    lax.fori_loop(0, nslc, slice_pass, 0)


def sc_scatter_add(d2d, u_flat, m_rows, gp, nslc):
    kfn = pl.kernel(
        functools.partial(_scatter_body, m_rows=m_rows, gp=gp, nslc=nslc),
        mesh=plsc.VectorSubcoreMesh(**_MESH),
        out_type=jax.ShapeDtypeStruct((2, gp, nslc * 16), jnp.float32),
        scratch_types=[
            pltpu.VMEM((8, 128), jnp.int32),
            pltpu.VMEM((8, 128), jnp.int32),
            pltpu.VMEM((1024, 16), jnp.float32),
            pltpu.VMEM((800, 16), jnp.float32),
            pltpu.VMEM_SHARED((gp, 16), jnp.float32),
            pltpu.SemaphoreType.DMA,
        ],
    )
    return kfn(d2d, u_flat)


def _padw(w):
    k = w.shape[0]
    return jnp.zeros((k, P), jnp.float32).at[:, :100].set(w)


def _padb(b, bias_one):
    bp = jnp.zeros((8, P), jnp.float32).at[0, :100].set(b)
    if bias_one:
        bp = bp.at[0, 111].set(1.0)
    return bp


def _gat_layer(xin, s2d, d2d, ze, Wl, bl, Wr, br, attp, biasp, relu):
    xl = tc_matmul(xin, _padw(Wl), _padb(bl, True), 800)
    xr = tc_matmul(xin, _padw(Wr), _padb(br, True), 800)
    xls, msum = sc_gather_add(xl, xr, ze, s2d, d2d, M_PAD)
    alpha = tc_alpha(msum, attp, 1024)
    cmax = tc_maxred(alpha)
    u = tc_u(alpha, cmax, xls, 1024, E + N)
    p_num = sc_scatter_add(d2d, u.reshape(M_PAD * 7, 16), M_PAD, GP_N, 7)
    return tc_h(p_num, biasp, relu)


def kernel(x, edge_index, edge_attr, batch, Wl1, bl1, Wr1, br1, We1, att1,
           bias1, Wl2, bl2, Wr2, br2, We2, att2, bias2, Wlin, blin):
    src = edge_index[0].astype(jnp.int32)
    dst = edge_index[1].astype(jnp.int32)
    ar = jnp.arange(N, dtype=jnp.int32)
    pad_m = jnp.zeros((M_PAD - E - N,), jnp.int32)
    s2d = jnp.concatenate([src, ar, pad_m]).reshape(M_PAD // 128, 128)
    d2d = jnp.concatenate([dst, ar, pad_m]).reshape(M_PAD // 128, 128)

    ea_aug = jnp.zeros((E_PAD, 48), jnp.float32)
    ea_aug = ea_aug.at[:E, :32].set(edge_attr).at[:E, 47].set(1.0)
    dstp = jnp.concatenate([dst, jnp.zeros((E_PAD - E,), jnp.int32)])
    p_attr = sc_scatter_add(dstp.reshape(E_PAD // 128, 128),
                            ea_aug.reshape(E_PAD * 3, 16), E_PAD, GP_N, 3)
    loop_attr = tc_loopattr(p_attr)

    ea_all = jnp.concatenate(
        [edge_attr, loop_attr, jnp.zeros((M_PAD - E - N, 32), jnp.float32)])
    ze1 = tc_matmul(ea_all, _padw(We1), jnp.zeros((8, P), jnp.float32), 1024)
    ze2 = tc_matmul(ea_all, _padw(We2), jnp.zeros((8, P), jnp.float32), 1024)

    att1p = jnp.zeros((8, P), jnp.float32).at[0, :100].set(att1)
    att2p = jnp.zeros((8, P), jnp.float32).at[0, :100].set(att2)

    h1 = _gat_layer(x, s2d, d2d, ze1, Wl1, bl1, Wr1, br1, att1p,
                    _padb(bias1, False), relu=True)
    h2 = _gat_layer(h1, s2d, d2d, ze2, Wl2, bl2, Wr2, br2, att2p,
                    _padb(bias2, False), relu=False)

    h2p = jnp.zeros((NP_PAD, P), jnp.float32).at[:N].set(h2)
    bat = jnp.concatenate(
        [batch.astype(jnp.int32), jnp.zeros((NP_PAD - N,), jnp.int32)])
    p_pool = sc_scatter_add(bat.reshape(NP_PAD // 128, 128),
                            h2p.reshape(NP_PAD * 7, 16), NP_PAD, GP_B, 7)

    wlinp = jnp.zeros((P, 8), jnp.float32).at[:100, 0].set(Wlin[:, 0])
    blinp = jnp.broadcast_to(blin.reshape(1, 1), (8, 8)).astype(jnp.float32)
    out = tc_head(p_pool, wlinp, blinp)
    return out[:, :1]


# speedup vs baseline: 1.2607x; 1.2607x over previous
import functools

import jax
import jax.numpy as jnp
from jax import lax
from jax.experimental import pallas as pl
from jax.experimental.pallas import tpu as pltpu
from jax.experimental.pallas import tpu_sc as plsc

N = 100000
E = 1600000
B = 128
P = 128
NW = 32
M_PAD = 1703936
E_PAD = 1601536
NP_PAD = 102400
GP_N = 102400
GP_B = 1024

_MESH = dict(core_axis_name="c", subcore_axis_name="s")


def _mm_body(x_ref, w_ref, b_ref, o_ref):
    o_ref[...] = jnp.dot(x_ref[...], w_ref[...],
                         preferred_element_type=jnp.float32) + b_ref[0:1, :]


def tc_matmul(x, w, b, mblk):
    m, k = x.shape
    p = w.shape[1]
    assert m % mblk == 0
    return pl.pallas_call(
        _mm_body,
        grid=(m // mblk,),
        in_specs=[
            pl.BlockSpec((mblk, k), lambda i: (i, 0)),
            pl.BlockSpec((k, p), lambda i: (0, 0)),
            pl.BlockSpec((8, p), lambda i: (0, 0)),
        ],
        out_specs=pl.BlockSpec((mblk, p), lambda i: (i, 0)),
        out_shape=jax.ShapeDtypeStruct((m, p), jnp.float32),
    )(x, w, b)


def _loopattr_body(pa_ref, o_ref):
    q = pa_ref[0] + pa_ref[1]
    deg = jnp.maximum(q[:, 32:33], 1.0)
    o_ref[...] = q[:, :32] / deg


def tc_loopattr(p_attr):
    mblk = 800
    return pl.pallas_call(
        _loopattr_body,
        grid=(N // mblk,),
        in_specs=[pl.BlockSpec((2, mblk, P), lambda i: (0, i, 0))],
        out_specs=pl.BlockSpec((mblk, 32), lambda i: (i, 0)),
        out_shape=jax.ShapeDtypeStruct((N, 32), jnp.float32),
    )(p_attr)


def _alpha_body(ms_ref, att_ref, o_ref):
    m = ms_ref[...]
    m = jnp.where(m >= 0, m, 0.2 * m)
    o_ref[...] = jnp.sum(m * att_ref[0:1, :], axis=1)


def tc_alpha(msum, attp, mblk):
    m = msum.shape[0]
    return pl.pallas_call(
        _alpha_body,
        grid=(m // mblk,),
        in_specs=[
            pl.BlockSpec((mblk, P), lambda i: (i, 0)),
            pl.BlockSpec((8, P), lambda i: (0, 0)),
        ],
        out_specs=pl.BlockSpec((mblk,), lambda i: (i,)),
        out_shape=jax.ShapeDtypeStruct((m,), jnp.float32),
    )(msum, attp)


def _maxred_body(a_ref, o_ref):
    o_ref[...] = jnp.full((8,), jnp.max(a_ref[...]), jnp.float32)


def tc_maxred(alpha):
    m = alpha.shape[0]
    return pl.pallas_call(
        _maxred_body,
        in_specs=[pl.BlockSpec((m,), lambda: (0,))],
        out_specs=pl.BlockSpec((8,), lambda: (0,)),
        out_shape=jax.ShapeDtypeStruct((8,), jnp.float32),
    )(alpha)


def _u_body(a_ref, c_ref, xls_ref, o_ref, *, mblk, m_real):
    i = pl.program_id(0)
    rows = i * mblk + lax.broadcasted_iota(jnp.int32, (mblk,), 0)
    w = jnp.exp(a_ref[...] - c_ref[0])
    w = jnp.where(rows < m_real, w, 0.0)
    o_ref[...] = w[:, None] * xls_ref[...]


def tc_u(alpha, cmax, xls, mblk, m_real):
    m = alpha.shape[0]
    return pl.pallas_call(
        functools.partial(_u_body, mblk=mblk, m_real=m_real),
        grid=(m // mblk,),
        in_specs=[
            pl.BlockSpec((mblk,), lambda i: (i,)),
            pl.BlockSpec((8,), lambda i: (0,)),
            pl.BlockSpec((mblk, P), lambda i: (i, 0)),
        ],
        out_specs=pl.BlockSpec((mblk, P), lambda i: (i, 0)),
        out_shape=jax.ShapeDtypeStruct((m, P), jnp.float32),
    )(alpha, cmax, xls)


def _h_body(pn_ref, b_ref, o_ref, *, relu):
    q = pn_ref[0] + pn_ref[1]
    inv = 1.0 / jnp.maximum(q[:, 127:128], 1e-16)
    h = q * inv + b_ref[0:1, :]
    if relu:
        h = jnp.maximum(h, 0.0)
    o_ref[...] = h


def tc_h(p_num, biasp, relu):
    mblk = 800
    return pl.pallas_call(
        functools.partial(_h_body, relu=relu),
        grid=(N // mblk,),
        in_specs=[
            pl.BlockSpec((2, mblk, P), lambda i: (0, i, 0)),
            pl.BlockSpec((8, P), lambda i: (0, 0)),
        ],
        out_specs=pl.BlockSpec((mblk, P), lambda i: (i, 0)),
        out_shape=jax.ShapeDtypeStruct((N, P), jnp.float32),
    )(p_num, biasp)


def _head_body(pp_ref, w_ref, b_ref, o_ref):
    q = pp_ref[0, :B, :] + pp_ref[1, :B, :]
    cnt = jnp.maximum(q[:, 127:128], 1.0)
    out = jnp.dot(q / cnt, w_ref[...], preferred_element_type=jnp.float32)
    o_ref[...] = jax.nn.sigmoid(out + b_ref[0:1, :])


def tc_head(p_pool, wlinp, blinp):
    return pl.pallas_call(
        _head_body,
        in_specs=[
            pl.BlockSpec((2, GP_B, P), lambda: (0, 0, 0)),
            pl.BlockSpec((P, 8), lambda: (0, 0)),
            pl.BlockSpec((8, 8), lambda: (0, 0)),
        ],
        out_specs=pl.BlockSpec((B, 8), lambda: (0, 0)),
        out_shape=jax.ShapeDtypeStruct((B, 8), jnp.float32),
    )(p_pool, wlinp, blinp)


def _gather_body(xl_hbm, xr_hbm, ze_hbm, s_hbm, d_hbm, xls_hbm, msum_hbm,
                 sbuf, dbuf, xlb, xrb, zeb, sem, *, m_rows):
    wid = lax.axis_index("s") * 2 + lax.axis_index("c")
    pw = m_rows // NW
    sup = 256
    nsup = pw // sup
    base0 = wid * pw

    def step(t, _):
        base = base0 + t * sup
        pltpu.sync_copy(s_hbm.at[pl.ds(base, 128)], sbuf.at[0])
        pltpu.sync_copy(s_hbm.at[pl.ds(base + 128, 128)], sbuf.at[1])
        pltpu.sync_copy(d_hbm.at[pl.ds(base, 128)], dbuf.at[0])
        pltpu.sync_copy(d_hbm.at[pl.ds(base + 128, 128)], dbuf.at[1])
        cps = [
            pltpu.async_copy(xl_hbm.at[sbuf.at[0]], xlb.at[pl.ds(0, 128)], sem),
            pltpu.async_copy(xl_hbm.at[sbuf.at[1]], xlb.at[pl.ds(128, 128)], sem),
            pltpu.async_copy(xr_hbm.at[dbuf.at[0]], xrb.at[pl.ds(0, 128)], sem),
            pltpu.async_copy(xr_hbm.at[dbuf.at[1]], xrb.at[pl.ds(128, 128)], sem),
            pltpu.async_copy(ze_hbm.at[pl.ds(base, sup)], zeb, sem),
        ]
        for cp in cps:
            cp.wait()
        pltpu.sync_copy(xlb, xls_hbm.at[pl.ds(base, sup)])

        def row(r, _):
            for c in range(8):
                sl = pl.ds(c * 16, 16)
                xlb[r, sl] = xlb[r, sl] + xrb[r, sl] + zeb[r, sl]
            return 0

        lax.fori_loop(0, sup, row, 0)
        pltpu.sync_copy(xlb, msum_hbm.at[pl.ds(base, sup)])
        return 0

    lax.fori_loop(0, nsup, step, 0)


def sc_gather_add(xl, xr, ze, s1d, d1d, m_rows):
    kfn = pl.kernel(
        functools.partial(_gather_body, m_rows=m_rows),
        mesh=plsc.VectorSubcoreMesh(**_MESH),
        out_type=[
            jax.ShapeDtypeStruct((m_rows, P), jnp.float32),
            jax.ShapeDtypeStruct((m_rows, P), jnp.float32),
        ],
        scratch_types=[
            pltpu.VMEM((2, 128), jnp.int32),
            pltpu.VMEM((2, 128), jnp.int32),
            pltpu.VMEM((256, P), jnp.float32),
            pltpu.VMEM((256, P), jnp.float32),
            pltpu.VMEM((256, P), jnp.float32),
            pltpu.SemaphoreType.DMA,
        ],
        compiler_params=pltpu.CompilerParams(needs_layout_passes=False),
    )
    return kfn(xl, xr, ze, s1d, d1d)


def _scatter_body(d2_hbm, u_hbm, out_hbm, dbuf, idb, ddb, dd2, ubuf, zbuf,
                  acc, sem, *, m_rows, gp, nr):
    cid = lax.axis_index("c")
    sid = lax.axis_index("s")
    wid = sid * 2 + cid
    tot_ch = m_rows // 1024
    c0 = wid * tot_ch // NW
    c1 = (wid + 1) * tot_ch // NW
    nranges = gp // nr
    zr = nr // 16
    iota16 = lax.iota(jnp.int32, 16)

    def fill_zbuf(r, _):
        for c in range(8):
            zbuf[r, pl.ds(c * 16, 16)] = jnp.zeros((16,), jnp.float32)
        return 0

    lax.fori_loop(0, 200, fill_zbuf, 0)

    def compact_chunk(cb, lo):
        pltpu.sync_copy(d2_hbm.at[pl.ds(pl.multiple_of(cb // 128, 8), 8), :],
                        dbuf)

        def sub(q, cnt):
            dv = dbuf[q // 8, pl.ds((q % 8) * 16, 16)]
            inr = (dv >= lo) & (dv < lo + nr)
            cs = plsc.cumsum(jnp.where(inr, 1, 0).astype(jnp.int32))
            pos = cnt + cs - 1
            plsc.store_scatter(idb, [pos], cb + q * 16 + iota16, mask=inr)
            plsc.store_scatter(ddb, [pos], dv - lo, mask=inr)
            return cnt + jnp.max(cs)

        return lax.fori_loop(0, 64, sub, 0)

    def drain(cnt):
        def batch(b, _):
            off = b * 128
            for k in range(8):
                sl = pl.ds(off + k * 16, 16)
                lanepos = off + k * 16 + iota16
                keep = lanepos < cnt
                idv = jnp.where(keep, idb[sl], m_rows - 1)
                ddv = jnp.where(keep, ddb[sl], nr)
                idb[sl] = idv
                dd2[0, pl.ds(k * 16, 16)] = ddv
            pltpu.async_copy(u_hbm.at[idb.at[pl.ds(off, 128)]], ubuf,
                             sem).wait()
            pltpu.sync_copy(ubuf, acc.at[dd2.at[0]], add=True)
            return 0

        lax.fori_loop(0, (cnt + 127) // 128, batch, 0)

    def range_pass(r, _):
        lo = r * nr
        done = 0
        while done < zr:
            cz = min(200, zr - done)
            pltpu.sync_copy(zbuf.at[pl.ds(0, cz)],
                            acc.at[pl.ds(sid * zr + done, cz)])
            done += cz
        plsc.subcore_barrier()

        def chunk(t, _):
            drain(compact_chunk(t * 1024, lo))
            return 0

        lax.fori_loop(c0, c1, chunk, 0)
        plsc.subcore_barrier()
        pltpu.sync_copy(acc.at[pl.ds(sid * zr, zr)],
                        out_hbm.at[cid, pl.ds(r * nr + sid * zr, zr), :])
        plsc.subcore_barrier()
        return 0

    lax.fori_loop(0, nranges, range_pass, 0)


def sc_scatter_add(d1d, u, m_rows, gp, nr):
    kfn = pl.kernel(
        functools.partial(_scatter_body, m_rows=m_rows, gp=gp, nr=nr),
        mesh=plsc.VectorSubcoreMesh(**_MESH),
        out_type=jax.ShapeDtypeStruct((2, gp, P), jnp.float32),
        scratch_types=[
            pltpu.VMEM((8, 128), jnp.int32),
            pltpu.VMEM((1024,), jnp.int32),
            pltpu.VMEM((1024,), jnp.int32),
            pltpu.VMEM((1, 128), jnp.int32),
            pltpu.VMEM((128, P), jnp.float32),
            pltpu.VMEM((200, P), jnp.float32),
            pltpu.VMEM_SHARED((nr + 16, P), jnp.float32),
            pltpu.SemaphoreType.DMA,
        ],
        compiler_params=pltpu.CompilerParams(needs_layout_passes=False),
    )
    return kfn(d1d.reshape(m_rows // 128, 128), u)


def _padw(w, kp=None):
    k = w.shape[0]
    kp = k if kp is None else kp
    return jnp.zeros((kp, P), jnp.float32).at[:k, :100].set(w)


def _padb(b, bias_one):
    bp = jnp.zeros((8, P), jnp.float32).at[0, :100].set(b)
    if bias_one:
        bp = bp.at[0, 127].set(1.0)
    return bp


def _gat_layer(xin, s1d, d1d, ze, Wl, bl, Wr, br, attp, biasp, relu):
    kp = xin.shape[1]
    xl = tc_matmul(xin, _padw(Wl, kp), _padb(bl, True), 800)
    xr = tc_matmul(xin, _padw(Wr, kp), _padb(br, True), 800)
    xls, msum = sc_gather_add(xl, xr, ze, s1d, d1d, M_PAD)
    alpha = tc_alpha(msum, attp, 1024)
    cmax = tc_maxred(alpha)
    u = tc_u(alpha, cmax, xls, 1024, E + N)
    p_num = sc_scatter_add(d1d, u, M_PAD, GP_N, 10240)
    return tc_h(p_num, biasp, relu)


def kernel(x, edge_index, edge_attr, batch, Wl1, bl1, Wr1, br1, We1, att1,
           bias1, Wl2, bl2, Wr2, br2, We2, att2, bias2, Wlin, blin):
    src = edge_index[0].astype(jnp.int32)
    dst = edge_index[1].astype(jnp.int32)
    ar = jnp.arange(N, dtype=jnp.int32)
    pad_m = jnp.zeros((M_PAD - E - N,), jnp.int32)
    s1d = jnp.concatenate([src, ar, pad_m])
    d1d = jnp.concatenate([dst, ar, pad_m])

    ea_aug = jnp.zeros((E_PAD, P), jnp.float32)
    ea_aug = ea_aug.at[:E, :32].set(edge_attr).at[:E, 32].set(1.0)
    dstp = jnp.concatenate([dst, jnp.zeros((E_PAD - E,), jnp.int32)])
    p_attr = sc_scatter_add(dstp, ea_aug, E_PAD, GP_N, 10240)
    loop_attr = tc_loopattr(p_attr)

    ea_all = jnp.concatenate(
        [edge_attr, loop_attr, jnp.zeros((M_PAD - E - N, 32), jnp.float32)])
    ze1 = tc_matmul(ea_all, _padw(We1), jnp.zeros((8, P), jnp.float32), 1024)
    ze2 = tc_matmul(ea_all, _padw(We2), jnp.zeros((8, P), jnp.float32), 1024)

    att1p = jnp.zeros((8, P), jnp.float32).at[0, :100].set(att1)
    att2p = jnp.zeros((8, P), jnp.float32).at[0, :100].set(att2)

    h1 = _gat_layer(x, s1d, d1d, ze1, Wl1, bl1, Wr1, br1, att1p,
                    _padb(bias1, False), relu=True)
    h2 = _gat_layer(h1, s1d, d1d, ze2, Wl2, bl2, Wr2, br2, att2p,
                    _padb(bias2, False), relu=False)

    h2p = jnp.zeros((NP_PAD, P), jnp.float32).at[:N].set(h2)
    bat = jnp.concatenate(
        [batch.astype(jnp.int32), jnp.zeros((NP_PAD - N,), jnp.int32)])
    p_pool = sc_scatter_add(bat, h2p, NP_PAD, GP_B, 1024)

    wlinp = jnp.zeros((P, 8), jnp.float32).at[:100, 0].set(Wlin[:, 0])
    blinp = jnp.broadcast_to(blin.reshape(1, 1), (8, 8)).astype(jnp.float32)
    out = tc_head(p_pool, wlinp, blinp)
    return out[:, :1]


# speedup vs baseline: 1.2624x; 1.0014x over previous
import functools

import jax
import jax.numpy as jnp
from jax import lax
from jax.experimental import pallas as pl
from jax.experimental.pallas import tpu as pltpu
from jax.experimental.pallas import tpu_sc as plsc

N = 100000
E = 1600000
B = 128
P = 128
NW = 32
M_PAD = 1703936
E_PAD = 1601536
NP_PAD = 102400
GP_N = 102400
GP_B = 1024

_MESH = dict(core_axis_name="c", subcore_axis_name="s")


def _mm_body(x_ref, w_ref, b_ref, o_ref):
    o_ref[...] = jnp.dot(x_ref[...], w_ref[...],
                         preferred_element_type=jnp.float32) + b_ref[0:1, :]


def tc_matmul(x, w, b, mblk):
    m, k = x.shape
    p = w.shape[1]
    assert m % mblk == 0
    return pl.pallas_call(
        _mm_body,
        grid=(m // mblk,),
        in_specs=[
            pl.BlockSpec((mblk, k), lambda i: (i, 0)),
            pl.BlockSpec((k, p), lambda i: (0, 0)),
            pl.BlockSpec((8, p), lambda i: (0, 0)),
        ],
        out_specs=pl.BlockSpec((mblk, p), lambda i: (i, 0)),
        out_shape=jax.ShapeDtypeStruct((m, p), jnp.float32),
    )(x, w, b)


def _loopattr_body(pa_ref, o_ref):
    q = pa_ref[...]
    deg = jnp.maximum(q[:, 32:33], 1.0)
    o_ref[...] = q[:, :32] / deg


def tc_loopattr(p_attr):
    mblk = 800
    return pl.pallas_call(
        _loopattr_body,
        grid=(N // mblk,),
        in_specs=[pl.BlockSpec((mblk, P), lambda i: (i, 0))],
        out_specs=pl.BlockSpec((mblk, 32), lambda i: (i, 0)),
        out_shape=jax.ShapeDtypeStruct((N, 32), jnp.float32),
    )(p_attr)


def _alpha_body(ms_ref, att_ref, o_ref):
    m = ms_ref[...]
    m = jnp.where(m >= 0, m, 0.2 * m)
    o_ref[...] = jnp.sum(m * att_ref[0:1, :], axis=1)


def tc_alpha(msum, attp, mblk):
    m = msum.shape[0]
    return pl.pallas_call(
        _alpha_body,
        grid=(m // mblk,),
        in_specs=[
            pl.BlockSpec((mblk, P), lambda i: (i, 0)),
            pl.BlockSpec((8, P), lambda i: (0, 0)),
        ],
        out_specs=pl.BlockSpec((mblk,), lambda i: (i,)),
        out_shape=jax.ShapeDtypeStruct((m,), jnp.float32),
    )(msum, attp)


def _maxred_body(a_ref, o_ref):
    o_ref[...] = jnp.full((8,), jnp.max(a_ref[...]), jnp.float32)


def tc_maxred(alpha):
    m = alpha.shape[0]
    return pl.pallas_call(
        _maxred_body,
        in_specs=[pl.BlockSpec((m,), lambda: (0,))],
        out_specs=pl.BlockSpec((8,), lambda: (0,)),
        out_shape=jax.ShapeDtypeStruct((8,), jnp.float32),
    )(alpha)


def _u_body(a_ref, c_ref, xls_ref, o_ref, *, mblk, m_real):
    i = pl.program_id(0)
    rows = i * mblk + lax.broadcasted_iota(jnp.int32, (mblk,), 0)
    w = jnp.exp(a_ref[...] - c_ref[0])
    w = jnp.where(rows < m_real, w, 0.0)
    o_ref[...] = w[:, None] * xls_ref[...]


def tc_u(alpha, cmax, xls, mblk, m_real):
    m = alpha.shape[0]
    return pl.pallas_call(
        functools.partial(_u_body, mblk=mblk, m_real=m_real),
        grid=(m // mblk,),
        in_specs=[
            pl.BlockSpec((mblk,), lambda i: (i,)),
            pl.BlockSpec((8,), lambda i: (0,)),
            pl.BlockSpec((mblk, P), lambda i: (i, 0)),
        ],
        out_specs=pl.BlockSpec((mblk, P), lambda i: (i, 0)),
        out_shape=jax.ShapeDtypeStruct((m, P), jnp.float32),
    )(alpha, cmax, xls)


def _h_body(pn_ref, b_ref, o_ref, *, relu):
    q = pn_ref[...]
    inv = 1.0 / jnp.maximum(q[:, 127:128], 1e-16)
    h = q * inv + b_ref[0:1, :]
    if relu:
        h = jnp.maximum(h, 0.0)
    o_ref[...] = h


def tc_h(p_num, biasp, relu):
    mblk = 800
    return pl.pallas_call(
        functools.partial(_h_body, relu=relu),
        grid=(N // mblk,),
        in_specs=[
            pl.BlockSpec((mblk, P), lambda i: (i, 0)),
            pl.BlockSpec((8, P), lambda i: (0, 0)),
        ],
        out_specs=pl.BlockSpec((mblk, P), lambda i: (i, 0)),
        out_shape=jax.ShapeDtypeStruct((N, P), jnp.float32),
    )(p_num, biasp)


def _head_body(pp_ref, w_ref, b_ref, o_ref):
    q = pp_ref[:B, :]
    cnt = jnp.maximum(q[:, 127:128], 1.0)
    out = jnp.dot(q / cnt, w_ref[...], preferred_element_type=jnp.float32)
    o_ref[...] = jax.nn.sigmoid(out + b_ref[0:1, :])


def tc_head(p_pool, wlinp, blinp):
    return pl.pallas_call(
        _head_body,
        in_specs=[
            pl.BlockSpec((GP_B, P), lambda: (0, 0)),
            pl.BlockSpec((P, 8), lambda: (0, 0)),
            pl.BlockSpec((8, 8), lambda: (0, 0)),
        ],
        out_specs=pl.BlockSpec((B, 8), lambda: (0, 0)),
        out_shape=jax.ShapeDtypeStruct((B, 8), jnp.float32),
    )(p_pool, wlinp, blinp)


def _gather_body(xl_hbm, xr_hbm, ze_hbm, s_hbm, d_hbm, xls_hbm, msum_hbm,
                 sbuf, dbuf, xlb, xrb, zeb, sem, *, m_rows):
    wid = lax.axis_index("s") * 2 + lax.axis_index("c")
    pw = m_rows // NW
    sup = 256
    nsup = pw // sup
    base0 = wid * pw

    def step(t, _):
        base = base0 + t * sup
        pltpu.sync_copy(s_hbm.at[pl.ds(base, 128)], sbuf.at[0])
        pltpu.sync_copy(s_hbm.at[pl.ds(base + 128, 128)], sbuf.at[1])
        pltpu.sync_copy(d_hbm.at[pl.ds(base, 128)], dbuf.at[0])
        pltpu.sync_copy(d_hbm.at[pl.ds(base + 128, 128)], dbuf.at[1])
        cps = [
            pltpu.async_copy(xl_hbm.at[sbuf.at[0]], xlb.at[pl.ds(0, 128)], sem),
            pltpu.async_copy(xl_hbm.at[sbuf.at[1]], xlb.at[pl.ds(128, 128)], sem),
            pltpu.async_copy(xr_hbm.at[dbuf.at[0]], xrb.at[pl.ds(0, 128)], sem),
            pltpu.async_copy(xr_hbm.at[dbuf.at[1]], xrb.at[pl.ds(128, 128)], sem),
            pltpu.async_copy(ze_hbm.at[pl.ds(base, sup)], zeb, sem),
        ]
        for cp in cps:
            cp.wait()
        pltpu.sync_copy(xlb, xls_hbm.at[pl.ds(base, sup)])

        def row(r, _):
            for c in range(8):
                sl = pl.ds(c * 16, 16)
                xlb[r, sl] = xlb[r, sl] + xrb[r, sl] + zeb[r, sl]
            return 0

        lax.fori_loop(0, sup, row, 0)
        pltpu.sync_copy(xlb, msum_hbm.at[pl.ds(base, sup)])
        return 0

    lax.fori_loop(0, nsup, step, 0)


def sc_gather_add(xl, xr, ze, s1d, d1d, m_rows):
    kfn = pl.kernel(
        functools.partial(_gather_body, m_rows=m_rows),
        mesh=plsc.VectorSubcoreMesh(**_MESH),
        out_type=[
            jax.ShapeDtypeStruct((m_rows, P), jnp.float32),
            jax.ShapeDtypeStruct((m_rows, P), jnp.float32),
        ],
        scratch_types=[
            pltpu.VMEM((2, 128), jnp.int32),
            pltpu.VMEM((2, 128), jnp.int32),
            pltpu.VMEM((256, P), jnp.float32),
            pltpu.VMEM((256, P), jnp.float32),
            pltpu.VMEM((256, P), jnp.float32),
            pltpu.SemaphoreType.DMA,
        ],
        compiler_params=pltpu.CompilerParams(needs_layout_passes=False),
    )
    return kfn(xl, xr, ze, s1d, d1d)


def _scatter_body(d2_hbm, u_hbm, out_hbm, dbuf, idb, ddb, dd2, ubuf,
                  acc, sem, *, m_rows, gp, nr, zrow0):
    cid = lax.axis_index("c")
    sid = lax.axis_index("s")
    tot_ch = m_rows // 1024
    c0 = sid * tot_ch // 16
    c1 = (sid + 1) * tot_ch // 16
    gph = gp // 2
    nranges = gph // nr
    zr = nr // 16
    iota16 = lax.iota(jnp.int32, 16)

    def compact_chunk(cb, lo):
        pltpu.sync_copy(d2_hbm.at[pl.ds(pl.multiple_of(cb // 128, 8), 8), :],
                        dbuf)

        def sub(q, cnt):
            dv = dbuf[q // 8, pl.ds((q % 8) * 16, 16)]
            inr = (dv >= lo) & (dv < lo + nr)
            cs = plsc.cumsum(jnp.where(inr, 1, 0).astype(jnp.int32))
            pos = cnt + cs - 1
            plsc.store_scatter(idb, [pos], cb + q * 16 + iota16, mask=inr)
            plsc.store_scatter(ddb, [pos], dv - lo, mask=inr)
            return cnt + jnp.max(cs)

        return lax.fori_loop(0, 64, sub, 0)

    def drain(cnt):
        nb = (cnt + 127) // 128

        def prep_fire(b, cur):
            off = b * 128
            for k in range(8):
                sl = pl.ds(off + k * 16, 16)
                lanepos = off + k * 16 + iota16
                keep = lanepos < cnt
                idb[sl] = jnp.where(keep, idb[sl], m_rows - 1)
                dd2[cur, pl.ds(k * 16, 16)] = jnp.where(keep, ddb[sl], nr)
            pltpu.async_copy(u_hbm.at[idb.at[pl.ds(off, 128)]],
                             ubuf.at[cur], sem)

        @pl.when(nb > 0)
        def _():
            prep_fire(0, 0)

        def batch(b, _):
            cur = b % 2

            @pl.when(b + 1 < nb)
            def _():
                prep_fire(b + 1, 1 - cur)

            pltpu.make_async_copy(u_hbm.at[idb.at[pl.ds(0, 128)]],
                                  ubuf.at[cur], sem).wait()
            pltpu.sync_copy(ubuf.at[cur], acc.at[dd2.at[cur]], add=True)
            return 0

        lax.fori_loop(0, nb, batch, 0)

    def range_pass(r, _):
        lo = cid * gph + r * nr
        done = 0
        while done < zr:
            cz = min(200, zr - done)
            pltpu.sync_copy(u_hbm.at[pl.ds(zrow0, cz), :],
                            acc.at[pl.ds(sid * zr + done, cz)])
            done += cz
        plsc.subcore_barrier()

        def chunk(t, _):
            drain(compact_chunk(t * 1024, lo))
            return 0

        lax.fori_loop(c0, c1, chunk, 0)
        plsc.subcore_barrier()
        pltpu.sync_copy(acc.at[pl.ds(sid * zr, zr)],
                        out_hbm.at[pl.ds(cid * gph + r * nr + sid * zr, zr),
                                   :])
        plsc.subcore_barrier()
        return 0

    lax.fori_loop(0, nranges, range_pass, 0)


def sc_scatter_add(d1d, u, m_rows, gp, nr, zrow0):
    kfn = pl.kernel(
        functools.partial(_scatter_body, m_rows=m_rows, gp=gp, nr=nr,
                          zrow0=zrow0),
        mesh=plsc.VectorSubcoreMesh(**_MESH),
        out_type=jax.ShapeDtypeStruct((gp, P), jnp.float32),
        scratch_types=[
            pltpu.VMEM((8, 128), jnp.int32),
            pltpu.VMEM((1024,), jnp.int32),
            pltpu.VMEM((1024,), jnp.int32),
            pltpu.VMEM((2, 128), jnp.int32),
            pltpu.VMEM((2, 128, P), jnp.float32),
            pltpu.VMEM_SHARED((nr + 16, P), jnp.float32),
            pltpu.SemaphoreType.DMA,
        ],
        compiler_params=pltpu.CompilerParams(needs_layout_passes=False),
    )
    return kfn(d1d.reshape(m_rows // 128, 128), u)


def _padw(w, kp=None):
    k = w.shape[0]
    kp = k if kp is None else kp
    return jnp.zeros((kp, P), jnp.float32).at[:k, :100].set(w)


def _padb(b, bias_one):
    bp = jnp.zeros((8, P), jnp.float32).at[0, :100].set(b)
    if bias_one:
        bp = bp.at[0, 127].set(1.0)
    return bp


def _gat_layer(xin, s1d, d1d, ze, Wl, bl, Wr, br, attp, biasp, relu):
    kp = xin.shape[1]
    xl = tc_matmul(xin, _padw(Wl, kp), _padb(bl, True), 800)
    xr = tc_matmul(xin, _padw(Wr, kp), _padb(br, True), 800)
    xls, msum = sc_gather_add(xl, xr, ze, s1d, d1d, M_PAD)
    alpha = tc_alpha(msum, attp, 1024)
    cmax = tc_maxred(alpha)
    u = tc_u(alpha, cmax, xls, 1024, E + N)
    p_num = sc_scatter_add(d1d, u, M_PAD, GP_N, 10240, E + N)
    return tc_h(p_num, biasp, relu)


def kernel(x, edge_index, edge_attr, batch, Wl1, bl1, Wr1, br1, We1, att1,
           bias1, Wl2, bl2, Wr2, br2, We2, att2, bias2, Wlin, blin):
    src = edge_index[0].astype(jnp.int32)
    dst = edge_index[1].astype(jnp.int32)
    ar = jnp.arange(N, dtype=jnp.int32)
    pad_m = jnp.zeros((M_PAD - E - N,), jnp.int32)
    s1d = jnp.concatenate([src, ar, pad_m])
    d1d = jnp.concatenate([dst, ar, pad_m])

    ea_aug = jnp.zeros((E_PAD, P), jnp.float32)
    ea_aug = ea_aug.at[:E, :32].set(edge_attr).at[:E, 32].set(1.0)
    dstp = jnp.concatenate([dst, jnp.zeros((E_PAD - E,), jnp.int32)])
    p_attr = sc_scatter_add(dstp, ea_aug, E_PAD, GP_N, 10240, E)
    loop_attr = tc_loopattr(p_attr)

    ea_all = jnp.concatenate(
        [edge_attr, loop_attr, jnp.zeros((M_PAD - E - N, 32), jnp.float32)])
    ze1 = tc_matmul(ea_all, _padw(We1), jnp.zeros((8, P), jnp.float32), 1024)
    ze2 = tc_matmul(ea_all, _padw(We2), jnp.zeros((8, P), jnp.float32), 1024)

    att1p = jnp.zeros((8, P), jnp.float32).at[0, :100].set(att1)
    att2p = jnp.zeros((8, P), jnp.float32).at[0, :100].set(att2)

    h1 = _gat_layer(x, s1d, d1d, ze1, Wl1, bl1, Wr1, br1, att1p,
                    _padb(bias1, False), relu=True)
    h2 = _gat_layer(h1, s1d, d1d, ze2, Wl2, bl2, Wr2, br2, att2p,
                    _padb(bias2, False), relu=False)

    h2p = jnp.zeros((NP_PAD, P), jnp.float32).at[:N].set(h2)
    bat = jnp.concatenate(
        [batch.astype(jnp.int32), jnp.zeros((NP_PAD - N,), jnp.int32)])
    p_pool = sc_scatter_add(bat, h2p, NP_PAD, GP_B, 512, N)

    wlinp = jnp.zeros((P, 8), jnp.float32).at[:100, 0].set(Wlin[:, 0])
    blinp = jnp.broadcast_to(blin.reshape(1, 1), (8, 8)).astype(jnp.float32)
    out = tc_head(p_pool, wlinp, blinp)
    return out[:, :1]


# speedup vs baseline: 1.2629x; 1.0004x over previous
import functools

import jax
import jax.numpy as jnp
from jax import lax
from jax.experimental import pallas as pl
from jax.experimental.pallas import tpu as pltpu
from jax.experimental.pallas import tpu_sc as plsc

N = 100000
E = 1600000
B = 128
P = 128
NW = 32
M_PAD = 1703936
E_PAD = 1601536
NP_PAD = 102400
GP_N = 102400
GP_B = 1024

_MESH = dict(core_axis_name="c", subcore_axis_name="s")


def _mm_body(x_ref, w_ref, b_ref, o_ref):
    o_ref[...] = jnp.dot(x_ref[...], w_ref[...],
                         preferred_element_type=jnp.float32) + b_ref[0:1, :]


def tc_matmul(x, w, b, mblk):
    m, k = x.shape
    p = w.shape[1]
    assert m % mblk == 0
    return pl.pallas_call(
        _mm_body,
        grid=(m // mblk,),
        in_specs=[
            pl.BlockSpec((mblk, k), lambda i: (i, 0)),
            pl.BlockSpec((k, p), lambda i: (0, 0)),
            pl.BlockSpec((8, p), lambda i: (0, 0)),
        ],
        out_specs=pl.BlockSpec((mblk, p), lambda i: (i, 0)),
        out_shape=jax.ShapeDtypeStruct((m, p), jnp.float32),
    )(x, w, b)


def _loopattr_body(pa_ref, o_ref):
    q = pa_ref[...]
    deg = jnp.maximum(q[:, 32:33], 1.0)
    o_ref[...] = q[:, :32] / deg


def tc_loopattr(p_attr):
    mblk = 800
    return pl.pallas_call(
        _loopattr_body,
        grid=(N // mblk,),
        in_specs=[pl.BlockSpec((mblk, P), lambda i: (i, 0))],
        out_specs=pl.BlockSpec((mblk, 32), lambda i: (i, 0)),
        out_shape=jax.ShapeDtypeStruct((N, 32), jnp.float32),
    )(p_attr)


def _alpha_body(ms_ref, att_ref, o_ref):
    m = ms_ref[...]
    m = jnp.where(m >= 0, m, 0.2 * m)
    o_ref[...] = jnp.sum(m * att_ref[0:1, :], axis=1)


def tc_alpha(msum, attp, mblk):
    m = msum.shape[0]
    return pl.pallas_call(
        _alpha_body,
        grid=(m // mblk,),
        in_specs=[
            pl.BlockSpec((mblk, P), lambda i: (i, 0)),
            pl.BlockSpec((8, P), lambda i: (0, 0)),
        ],
        out_specs=pl.BlockSpec((mblk,), lambda i: (i,)),
        out_shape=jax.ShapeDtypeStruct((m,), jnp.float32),
    )(msum, attp)


def _maxred_body(a_ref, o_ref):
    o_ref[...] = jnp.full((8,), jnp.max(a_ref[...]), jnp.float32)


def tc_maxred(alpha):
    m = alpha.shape[0]
    return pl.pallas_call(
        _maxred_body,
        in_specs=[pl.BlockSpec((m,), lambda: (0,))],
        out_specs=pl.BlockSpec((8,), lambda: (0,)),
        out_shape=jax.ShapeDtypeStruct((8,), jnp.float32),
    )(alpha)


def _u_body(a_ref, c_ref, xls_ref, o_ref, *, mblk, m_real):
    i = pl.program_id(0)
    rows = i * mblk + lax.broadcasted_iota(jnp.int32, (mblk,), 0)
    w = jnp.exp(a_ref[...] - c_ref[0])
    w = jnp.where(rows < m_real, w, 0.0)
    o_ref[...] = w[:, None] * xls_ref[...]


def tc_u(alpha, cmax, xls, mblk, m_real):
    m = alpha.shape[0]
    return pl.pallas_call(
        functools.partial(_u_body, mblk=mblk, m_real=m_real),
        grid=(m // mblk,),
        in_specs=[
            pl.BlockSpec((mblk,), lambda i: (i,)),
            pl.BlockSpec((8,), lambda i: (0,)),
            pl.BlockSpec((mblk, P), lambda i: (i, 0)),
        ],
        out_specs=pl.BlockSpec((mblk, P), lambda i: (i, 0)),
        out_shape=jax.ShapeDtypeStruct((m, P), jnp.float32),
    )(alpha, cmax, xls)


def _h_body(pn_ref, b_ref, o_ref, *, relu):
    q = pn_ref[...]
    inv = 1.0 / jnp.maximum(q[:, 127:128], 1e-16)
    h = q * inv + b_ref[0:1, :]
    if relu:
        h = jnp.maximum(h, 0.0)
    o_ref[...] = h


def tc_h(p_num, biasp, relu):
    mblk = 800
    return pl.pallas_call(
        functools.partial(_h_body, relu=relu),
        grid=(N // mblk,),
        in_specs=[
            pl.BlockSpec((mblk, P), lambda i: (i, 0)),
            pl.BlockSpec((8, P), lambda i: (0, 0)),
        ],
        out_specs=pl.BlockSpec((mblk, P), lambda i: (i, 0)),
        out_shape=jax.ShapeDtypeStruct((N, P), jnp.float32),
    )(p_num, biasp)


def _head_body(pp_ref, w_ref, b_ref, o_ref):
    q = pp_ref[:B, :]
    cnt = jnp.maximum(q[:, 127:128], 1.0)
    out = jnp.dot(q / cnt, w_ref[...], preferred_element_type=jnp.float32)
    o_ref[...] = jax.nn.sigmoid(out + b_ref[0:1, :])


def tc_head(p_pool, wlinp, blinp):
    return pl.pallas_call(
        _head_body,
        in_specs=[
            pl.BlockSpec((GP_B, P), lambda: (0, 0)),
            pl.BlockSpec((P, 8), lambda: (0, 0)),
            pl.BlockSpec((8, 8), lambda: (0, 0)),
        ],
        out_specs=pl.BlockSpec((B, 8), lambda: (0, 0)),
        out_shape=jax.ShapeDtypeStruct((B, 8), jnp.float32),
    )(p_pool, wlinp, blinp)


def _gather_body(xl_hbm, xr_hbm, ze_hbm, s_hbm, d_hbm, xls_hbm, msum_hbm,
                 sbuf, dbuf, xlb, xrb, zeb, sem, *, m_rows):
    wid = lax.axis_index("s") * 2 + lax.axis_index("c")
    pw = m_rows // NW
    sup = 256
    nsup = pw // sup
    base0 = wid * pw

    def step(t, _):
        base = base0 + t * sup
        pltpu.sync_copy(s_hbm.at[pl.ds(base, 128)], sbuf.at[0])
        pltpu.sync_copy(s_hbm.at[pl.ds(base + 128, 128)], sbuf.at[1])
        pltpu.sync_copy(d_hbm.at[pl.ds(base, 128)], dbuf.at[0])
        pltpu.sync_copy(d_hbm.at[pl.ds(base + 128, 128)], dbuf.at[1])
        cps = [
            pltpu.async_copy(xl_hbm.at[sbuf.at[0]], xlb.at[pl.ds(0, 128)], sem),
            pltpu.async_copy(xl_hbm.at[sbuf.at[1]], xlb.at[pl.ds(128, 128)], sem),
            pltpu.async_copy(xr_hbm.at[dbuf.at[0]], xrb.at[pl.ds(0, 128)], sem),
            pltpu.async_copy(xr_hbm.at[dbuf.at[1]], xrb.at[pl.ds(128, 128)], sem),
            pltpu.async_copy(ze_hbm.at[pl.ds(base, sup)], zeb, sem),
        ]
        for cp in cps:
            cp.wait()
        pltpu.sync_copy(xlb, xls_hbm.at[pl.ds(base, sup)])

        def row(r, _):
            for c in range(8):
                sl = pl.ds(c * 16, 16)
                xlb[r, sl] = xlb[r, sl] + xrb[r, sl] + zeb[r, sl]
            return 0

        lax.fori_loop(0, sup, row, 0)
        pltpu.sync_copy(xlb, msum_hbm.at[pl.ds(base, sup)])
        return 0

    lax.fori_loop(0, nsup, step, 0)


def sc_gather_add(xl, xr, ze, s1d, d1d, m_rows):
    kfn = pl.kernel(
        functools.partial(_gather_body, m_rows=m_rows),
        mesh=plsc.VectorSubcoreMesh(**_MESH),
        out_type=[
            jax.ShapeDtypeStruct((m_rows, P), jnp.float32),
            jax.ShapeDtypeStruct((m_rows, P), jnp.float32),
        ],
        scratch_types=[
            pltpu.VMEM((2, 128), jnp.int32),
            pltpu.VMEM((2, 128), jnp.int32),
            pltpu.VMEM((256, P), jnp.float32),
            pltpu.VMEM((256, P), jnp.float32),
            pltpu.VMEM((256, P), jnp.float32),
            pltpu.SemaphoreType.DMA,
        ],
        compiler_params=pltpu.CompilerParams(needs_layout_passes=False),
    )
    return kfn(xl, xr, ze, s1d, d1d)


def _scatter_body(d2_hbm, u_hbm, out_hbm, dbuf, idb, ddb, dd2, ubuf,
                  acc, sem, *, m_rows, gp, nr, zrow0):
    cid = lax.axis_index("c")
    sid = lax.axis_index("s")
    tot_ch = m_rows // 1024
    c0 = sid * tot_ch // 16
    c1 = (sid + 1) * tot_ch // 16
    gph = gp // 2
    nranges = gph // nr
    zr = nr // 16
    iota16 = lax.iota(jnp.int32, 16)

    def compact_chunk(cb, lo):
        pltpu.sync_copy(d2_hbm.at[pl.ds(pl.multiple_of(cb // 128, 8), 8), :],
                        dbuf)

        def sub(q, cnt):
            dv = dbuf[q // 8, pl.ds((q % 8) * 16, 16)]
            inr = (dv >= lo) & (dv < lo + nr)
            plsc.store_compressed(idb.at[pl.ds(cnt, 16)],
                                  cb + q * 16 + iota16, mask=inr)
            plsc.store_compressed(ddb.at[pl.ds(cnt, 16)], dv - lo, mask=inr)
            return cnt + jnp.max(plsc.all_reduce_population_count(inr))

        return lax.fori_loop(0, 64, sub, 0)

    def drain(cnt):
        nb = (cnt + 127) // 128

        def prep_fire(b, cur):
            off = b * 128
            for k in range(8):
                sl = pl.ds(off + k * 16, 16)
                lanepos = off + k * 16 + iota16
                keep = lanepos < cnt
                idb[sl] = jnp.where(keep, idb[sl], m_rows - 1)
                dd2[cur, pl.ds(k * 16, 16)] = jnp.where(keep, ddb[sl], nr)
            pltpu.async_copy(u_hbm.at[idb.at[pl.ds(off, 128)]],
                             ubuf.at[cur], sem)

        @pl.when(nb > 0)
        def _():
            prep_fire(0, 0)

        def batch(b, _):
            cur = b % 2

            @pl.when(b + 1 < nb)
            def _():
                prep_fire(b + 1, 1 - cur)

            pltpu.make_async_copy(u_hbm.at[idb.at[pl.ds(0, 128)]],
                                  ubuf.at[cur], sem).wait()
            pltpu.sync_copy(ubuf.at[cur], acc.at[dd2.at[cur]], add=True)
            return 0

        lax.fori_loop(0, nb, batch, 0)

    def range_pass(r, _):
        lo = cid * gph + r * nr
        done = 0
        while done < zr:
            cz = min(200, zr - done)
            pltpu.sync_copy(u_hbm.at[pl.ds(zrow0, cz), :],
                            acc.at[pl.ds(sid * zr + done, cz)])
            done += cz
        plsc.subcore_barrier()

        def chunk(t, _):
            drain(compact_chunk(t * 1024, lo))
            return 0

        lax.fori_loop(c0, c1, chunk, 0)
        plsc.subcore_barrier()
        pltpu.sync_copy(acc.at[pl.ds(sid * zr, zr)],
                        out_hbm.at[pl.ds(cid * gph + r * nr + sid * zr, zr),
                                   :])
        plsc.subcore_barrier()
        return 0

    lax.fori_loop(0, nranges, range_pass, 0)


def sc_scatter_add(d1d, u, m_rows, gp, nr, zrow0):
    kfn = pl.kernel(
        functools.partial(_scatter_body, m_rows=m_rows, gp=gp, nr=nr,
                          zrow0=zrow0),
        mesh=plsc.VectorSubcoreMesh(**_MESH),
        out_type=jax.ShapeDtypeStruct((gp, P), jnp.float32),
        scratch_types=[
            pltpu.VMEM((8, 128), jnp.int32),
            pltpu.VMEM((1040,), jnp.int32),
            pltpu.VMEM((1040,), jnp.int32),
            pltpu.VMEM((2, 128), jnp.int32),
            pltpu.VMEM((2, 128, P), jnp.float32),
            pltpu.VMEM_SHARED((nr + 16, P), jnp.float32),
            pltpu.SemaphoreType.DMA,
        ],
        compiler_params=pltpu.CompilerParams(needs_layout_passes=False),
    )
    return kfn(d1d.reshape(m_rows // 128, 128), u)


def _padw(w, kp=None):
    k = w.shape[0]
    kp = k if kp is None else kp
    return jnp.zeros((kp, P), jnp.float32).at[:k, :100].set(w)


def _padb(b, bias_one):
    bp = jnp.zeros((8, P), jnp.float32).at[0, :100].set(b)
    if bias_one:
        bp = bp.at[0, 127].set(1.0)
    return bp


def _gat_layer(xin, s1d, d1d, ze, Wl, bl, Wr, br, attp, biasp, relu):
    kp = xin.shape[1]
    xl = tc_matmul(xin, _padw(Wl, kp), _padb(bl, True), 800)
    xr = tc_matmul(xin, _padw(Wr, kp), _padb(br, True), 800)
    xls, msum = sc_gather_add(xl, xr, ze, s1d, d1d, M_PAD)
    alpha = tc_alpha(msum, attp, 1024)
    cmax = tc_maxred(alpha)
    u = tc_u(alpha, cmax, xls, 1024, E + N)
    p_num = sc_scatter_add(d1d, u, M_PAD, GP_N, 10240, E + N)
    return tc_h(p_num, biasp, relu)


def kernel(x, edge_index, edge_attr, batch, Wl1, bl1, Wr1, br1, We1, att1,
           bias1, Wl2, bl2, Wr2, br2, We2, att2, bias2, Wlin, blin):
    src = edge_index[0].astype(jnp.int32)
    dst = edge_index[1].astype(jnp.int32)
    ar = jnp.arange(N, dtype=jnp.int32)
    pad_m = jnp.zeros((M_PAD - E - N,), jnp.int32)
    s1d = jnp.concatenate([src, ar, pad_m])
    d1d = jnp.concatenate([dst, ar, pad_m])

    ea_aug = jnp.zeros((E_PAD, P), jnp.float32)
    ea_aug = ea_aug.at[:E, :32].set(edge_attr).at[:E, 32].set(1.0)
    dstp = jnp.concatenate([dst, jnp.zeros((E_PAD - E,), jnp.int32)])
    p_attr = sc_scatter_add(dstp, ea_aug, E_PAD, GP_N, 10240, E)
    loop_attr = tc_loopattr(p_attr)

    ea_all = jnp.concatenate(
        [edge_attr, loop_attr, jnp.zeros((M_PAD - E - N, 32), jnp.float32)])
    ze1 = tc_matmul(ea_all, _padw(We1), jnp.zeros((8, P), jnp.float32), 1024)
    ze2 = tc_matmul(ea_all, _padw(We2), jnp.zeros((8, P), jnp.float32), 1024)

    att1p = jnp.zeros((8, P), jnp.float32).at[0, :100].set(att1)
    att2p = jnp.zeros((8, P), jnp.float32).at[0, :100].set(att2)

    h1 = _gat_layer(x, s1d, d1d, ze1, Wl1, bl1, Wr1, br1, att1p,
                    _padb(bias1, False), relu=True)
    h2 = _gat_layer(h1, s1d, d1d, ze2, Wl2, bl2, Wr2, br2, att2p,
                    _padb(bias2, False), relu=False)

    h2p = jnp.zeros((NP_PAD, P), jnp.float32).at[:N].set(h2)
    bat = jnp.concatenate(
        [batch.astype(jnp.int32), jnp.zeros((NP_PAD - N,), jnp.int32)])
    p_pool = sc_scatter_add(bat, h2p, NP_PAD, GP_B, 512, N)

    wlinp = jnp.zeros((P, 8), jnp.float32).at[:100, 0].set(Wlin[:, 0])
    blinp = jnp.broadcast_to(blin.reshape(1, 1), (8, 8)).astype(jnp.float32)
    out = tc_head(p_pool, wlinp, blinp)
    return out[:, :1]


# speedup vs baseline: 3.6084x; 2.8571x over previous
import functools

import jax
import jax.numpy as jnp
from jax import lax
from jax.experimental import pallas as pl
from jax.experimental.pallas import tpu as pltpu
from jax.experimental.pallas import tpu_sc as plsc

N = 100000
E = 1600000
B = 128
P = 128
NW = 32
M_PAD = 1703936
E_PAD = 1601536
NP_PAD = 102400
GP_N = 102400
GP_B = 1024

_MESH = dict(core_axis_name="c", subcore_axis_name="s")


def _mm_body(x_ref, w_ref, b_ref, o_ref):
    o_ref[...] = jnp.dot(x_ref[...], w_ref[...],
                         preferred_element_type=jnp.float32) + b_ref[0:1, :]


def tc_matmul(x, w, b, mblk):
    m, k = x.shape
    p = w.shape[1]
    assert m % mblk == 0
    return pl.pallas_call(
        _mm_body,
        grid=(m // mblk,),
        in_specs=[
            pl.BlockSpec((mblk, k), lambda i: (i, 0)),
            pl.BlockSpec((k, p), lambda i: (0, 0)),
            pl.BlockSpec((8, p), lambda i: (0, 0)),
        ],
        out_specs=pl.BlockSpec((mblk, p), lambda i: (i, 0)),
        out_shape=jax.ShapeDtypeStruct((m, p), jnp.float32),
    )(x, w, b)


def _loopattr_body(pa_ref, o_ref):
    q = pa_ref[...]
    deg = jnp.maximum(q[:, 32:33], 1.0)
    o_ref[...] = q[:, :32] / deg


def tc_loopattr(p_attr):
    mblk = 800
    return pl.pallas_call(
        _loopattr_body,
        grid=(N // mblk,),
        in_specs=[pl.BlockSpec((mblk, P), lambda i: (i, 0))],
        out_specs=pl.BlockSpec((mblk, 32), lambda i: (i, 0)),
        out_shape=jax.ShapeDtypeStruct((N, 32), jnp.float32),
    )(p_attr)


def _alpha_body(ms_ref, att_ref, o_ref):
    m = ms_ref[...]
    m = jnp.where(m >= 0, m, 0.2 * m)
    o_ref[...] = jnp.sum(m * att_ref[0:1, :], axis=1)


def tc_alpha(msum, attp, mblk):
    m = msum.shape[0]
    return pl.pallas_call(
        _alpha_body,
        grid=(m // mblk,),
        in_specs=[
            pl.BlockSpec((mblk, P), lambda i: (i, 0)),
            pl.BlockSpec((8, P), lambda i: (0, 0)),
        ],
        out_specs=pl.BlockSpec((mblk,), lambda i: (i,)),
        out_shape=jax.ShapeDtypeStruct((m,), jnp.float32),
    )(msum, attp)


def _maxred_body(a_ref, o_ref):
    o_ref[...] = jnp.full((8,), jnp.max(a_ref[...]), jnp.float32)


def tc_maxred(alpha):
    m = alpha.shape[0]
    return pl.pallas_call(
        _maxred_body,
        in_specs=[pl.BlockSpec((m,), lambda: (0,))],
        out_specs=pl.BlockSpec((8,), lambda: (0,)),
        out_shape=jax.ShapeDtypeStruct((8,), jnp.float32),
    )(alpha)


def _u_body(a_ref, c_ref, xls_ref, o_ref, *, mblk, m_real):
    i = pl.program_id(0)
    rows = i * mblk + lax.broadcasted_iota(jnp.int32, (mblk,), 0)
    w = jnp.exp(a_ref[...] - c_ref[0])
    w = jnp.where(rows < m_real, w, 0.0)
    o_ref[...] = w[:, None] * xls_ref[...]


def tc_u(alpha, cmax, xls, mblk, m_real):
    m = alpha.shape[0]
    return pl.pallas_call(
        functools.partial(_u_body, mblk=mblk, m_real=m_real),
        grid=(m // mblk,),
        in_specs=[
            pl.BlockSpec((mblk,), lambda i: (i,)),
            pl.BlockSpec((8,), lambda i: (0,)),
            pl.BlockSpec((mblk, P), lambda i: (i, 0)),
        ],
        out_specs=pl.BlockSpec((mblk, P), lambda i: (i, 0)),
        out_shape=jax.ShapeDtypeStruct((m, P), jnp.float32),
    )(alpha, cmax, xls)


def _h_body(pn_ref, b_ref, o_ref, *, relu):
    q = pn_ref[...]
    inv = 1.0 / jnp.maximum(q[:, 127:128], 1e-16)
    h = q * inv + b_ref[0:1, :]
    if relu:
        h = jnp.maximum(h, 0.0)
    o_ref[...] = h


def tc_h(p_num, biasp, relu):
    mblk = 800
    return pl.pallas_call(
        functools.partial(_h_body, relu=relu),
        grid=(N // mblk,),
        in_specs=[
            pl.BlockSpec((mblk, P), lambda i: (i, 0)),
            pl.BlockSpec((8, P), lambda i: (0, 0)),
        ],
        out_specs=pl.BlockSpec((mblk, P), lambda i: (i, 0)),
        out_shape=jax.ShapeDtypeStruct((N, P), jnp.float32),
    )(p_num, biasp)


def _head_body(pp_ref, w_ref, b_ref, o_ref):
    q = pp_ref[:B, :]
    cnt = jnp.maximum(q[:, 127:128], 1.0)
    out = jnp.dot(q / cnt, w_ref[...], preferred_element_type=jnp.float32)
    o_ref[...] = jax.nn.sigmoid(out + b_ref[0:1, :])


def tc_head(p_pool, wlinp, blinp):
    return pl.pallas_call(
        _head_body,
        in_specs=[
            pl.BlockSpec((GP_B, P), lambda: (0, 0)),
            pl.BlockSpec((P, 8), lambda: (0, 0)),
            pl.BlockSpec((8, 8), lambda: (0, 0)),
        ],
        out_specs=pl.BlockSpec((B, 8), lambda: (0, 0)),
        out_shape=jax.ShapeDtypeStruct((B, 8), jnp.float32),
    )(p_pool, wlinp, blinp)


def _gather_body(xl_hbm, xr_hbm, ze_hbm, s_hbm, d_hbm, xls_hbm, msum_hbm,
                 sbuf, dbuf, xlb, xrb, zeb, sem, *, m_rows):
    wid = lax.axis_index("s") * 2 + lax.axis_index("c")
    pw = m_rows // NW
    sup = 256
    nsup = pw // sup
    base0 = wid * pw

    def step(t, _):
        base = base0 + t * sup
        pltpu.sync_copy(s_hbm.at[pl.ds(base, 128)], sbuf.at[0])
        pltpu.sync_copy(s_hbm.at[pl.ds(base + 128, 128)], sbuf.at[1])
        pltpu.sync_copy(d_hbm.at[pl.ds(base, 128)], dbuf.at[0])
        pltpu.sync_copy(d_hbm.at[pl.ds(base + 128, 128)], dbuf.at[1])
        cps = [
            pltpu.async_copy(xl_hbm.at[sbuf.at[0]], xlb.at[pl.ds(0, 128)], sem),
            pltpu.async_copy(xl_hbm.at[sbuf.at[1]], xlb.at[pl.ds(128, 128)], sem),
            pltpu.async_copy(xr_hbm.at[dbuf.at[0]], xrb.at[pl.ds(0, 128)], sem),
            pltpu.async_copy(xr_hbm.at[dbuf.at[1]], xrb.at[pl.ds(128, 128)], sem),
            pltpu.async_copy(ze_hbm.at[pl.ds(base, sup)], zeb, sem),
        ]
        for cp in cps:
            cp.wait()
        pltpu.sync_copy(xlb, xls_hbm.at[pl.ds(base, sup)])

        def row(r, _):
            for c in range(8):
                sl = pl.ds(c * 16, 16)
                xlb[r, sl] = xlb[r, sl] + xrb[r, sl] + zeb[r, sl]
            return 0

        lax.fori_loop(0, sup, row, 0)
        pltpu.sync_copy(xlb, msum_hbm.at[pl.ds(base, sup)])
        return 0

    lax.fori_loop(0, nsup, step, 0)


def sc_gather_add(xl, xr, ze, s1d, d1d, m_rows):
    kfn = pl.kernel(
        functools.partial(_gather_body, m_rows=m_rows),
        mesh=plsc.VectorSubcoreMesh(**_MESH),
        out_type=[
            jax.ShapeDtypeStruct((m_rows, P), jnp.float32),
            jax.ShapeDtypeStruct((m_rows, P), jnp.float32),
        ],
        scratch_types=[
            pltpu.VMEM((2, 128), jnp.int32),
            pltpu.VMEM((2, 128), jnp.int32),
            pltpu.VMEM((256, P), jnp.float32),
            pltpu.VMEM((256, P), jnp.float32),
            pltpu.VMEM((256, P), jnp.float32),
            pltpu.SemaphoreType.DMA,
        ],
        compiler_params=pltpu.CompilerParams(needs_layout_passes=False),
    )
    return kfn(xl, xr, ze, s1d, d1d)


def _scatter_body(d2_hbm, u_hbm, out_hbm, dbuf, idb, ddb, dd2, ubuf,
                  acc, sem, *, m_rows, gp, nr, zrow0):
    cid = lax.axis_index("c")
    sid = lax.axis_index("s")
    tot_ch = m_rows // 1024
    c0 = sid * tot_ch // 16
    c1 = (sid + 1) * tot_ch // 16
    gph = gp // 2
    nranges = gph // nr
    zr = nr // 16
    iota16 = lax.iota(jnp.int32, 16)

    def compact_chunk(cb, lo, cnt0):
        pltpu.sync_copy(d2_hbm.at[pl.ds(pl.multiple_of(cb // 128, 8), 8), :],
                        dbuf)

        def sub(q, cnt):
            dv = dbuf[q // 8, pl.ds((q % 8) * 16, 16)]
            inr = (dv >= lo) & (dv < lo + nr)
            plsc.store_compressed(idb.at[pl.ds(cnt, 16)],
                                  cb + q * 16 + iota16, mask=inr)
            plsc.store_compressed(ddb.at[pl.ds(cnt, 16)], dv - lo, mask=inr)
            return cnt + jnp.max(plsc.all_reduce_population_count(inr))

        return lax.fori_loop(0, 64, sub, cnt0)

    def drain(cnt, final):
        nb = (cnt + 127) // 128 if final else cnt // 128

        def prep_fire(b, cur):
            off = b * 128
            for k in range(8):
                sl = pl.ds(off + k * 16, 16)
                lanepos = off + k * 16 + iota16
                keep = lanepos < cnt
                idb[sl] = jnp.where(keep, idb[sl], m_rows - 1)
                dd2[cur, pl.ds(k * 16, 16)] = jnp.where(keep, ddb[sl], nr)
            pltpu.async_copy(u_hbm.at[idb.at[pl.ds(off, 128)]],
                             ubuf.at[cur], sem)

        @pl.when(nb > 0)
        def _():
            prep_fire(0, 0)

        def batch(b, _):
            cur = b % 2

            @pl.when(b + 1 < nb)
            def _():
                prep_fire(b + 1, 1 - cur)

            pltpu.make_async_copy(u_hbm.at[idb.at[pl.ds(0, 128)]],
                                  ubuf.at[cur], sem).wait()
            pltpu.sync_copy(ubuf.at[cur], acc.at[dd2.at[cur]], add=True)
            return 0

        lax.fori_loop(0, nb, batch, 0)
        rem = cnt - nb * 128

        @pl.when(rem > 0)
        def _():
            def mv(k, _):
                sl = pl.ds(nb * 128 + k * 16, 16)
                dl = pl.ds(k * 16, 16)
                idb[dl] = idb[sl]
                ddb[dl] = ddb[sl]
                return 0

            lax.fori_loop(0, 8, mv, 0)

        return rem

    def range_pass(r, _):
        lo = cid * gph + r * nr
        done = 0
        while done < zr:
            cz = min(200, zr - done)
            pltpu.sync_copy(u_hbm.at[pl.ds(zrow0, cz), :],
                            acc.at[pl.ds(sid * zr + done, cz)])
            done += cz
        plsc.subcore_barrier()

        def chunk(t, carry):
            return drain(compact_chunk(t * 1024, lo, carry), False)

        tailcnt = lax.fori_loop(c0, c1, chunk, 0)
        drain(tailcnt, True)
        plsc.subcore_barrier()
        pltpu.sync_copy(acc.at[pl.ds(sid * zr, zr)],
                        out_hbm.at[pl.ds(cid * gph + r * nr + sid * zr, zr),
                                   :])
        plsc.subcore_barrier()
        return 0

    lax.fori_loop(0, nranges, range_pass, 0)


def sc_scatter_add(d1d, u, m_rows, gp, nr, zrow0):
    kfn = pl.kernel(
        functools.partial(_scatter_body, m_rows=m_rows, gp=gp, nr=nr,
                          zrow0=zrow0),
        mesh=plsc.VectorSubcoreMesh(**_MESH),
        out_type=jax.ShapeDtypeStruct((gp, P), jnp.float32),
        scratch_types=[
            pltpu.VMEM((8, 128), jnp.int32),
            pltpu.VMEM((1184,), jnp.int32),
            pltpu.VMEM((1184,), jnp.int32),
            pltpu.VMEM((2, 128), jnp.int32),
            pltpu.VMEM((2, 128, P), jnp.float32),
            pltpu.VMEM_SHARED((nr + 16, P), jnp.float32),
            pltpu.SemaphoreType.DMA,
        ],
        compiler_params=pltpu.CompilerParams(needs_layout_passes=False),
    )
    return kfn(d1d.reshape(m_rows // 128, 128), u)


def _padw(w, kp=None):
    k = w.shape[0]
    kp = k if kp is None else kp
    return jnp.zeros((kp, P), jnp.float32).at[:k, :100].set(w)


def _padb(b, bias_one):
    bp = jnp.zeros((8, P), jnp.float32).at[0, :100].set(b)
    if bias_one:
        bp = bp.at[0, 127].set(1.0)
    return bp


def _gat_layer(xin, s1d, d1d, ze, Wl, bl, Wr, br, attp, biasp, relu):
    kp = xin.shape[1]
    xl = tc_matmul(xin, _padw(Wl, kp), _padb(bl, True), 800)
    xr = tc_matmul(xin, _padw(Wr, kp), _padb(br, True), 800)
    xls, msum = sc_gather_add(xl, xr, ze, s1d, d1d, M_PAD)
    alpha = tc_alpha(msum, attp, 1024)
    cmax = tc_maxred(alpha)
    u = tc_u(alpha, cmax, xls, 1024, E + N)
    p_num = sc_scatter_add(d1d, u, M_PAD, GP_N, 10240, E + N)
    return tc_h(p_num, biasp, relu)


def kernel(x, edge_index, edge_attr, batch, Wl1, bl1, Wr1, br1, We1, att1,
           bias1, Wl2, bl2, Wr2, br2, We2, att2, bias2, Wlin, blin):
    src = edge_index[0].astype(jnp.int32)
    dst = edge_index[1].astype(jnp.int32)
    ar = jnp.arange(N, dtype=jnp.int32)
    pad_m = jnp.zeros((M_PAD - E - N,), jnp.int32)
    s1d = jnp.concatenate([src, ar, pad_m])
    d1d = jnp.concatenate([dst, ar, pad_m])

    ea_aug = jnp.zeros((E_PAD, P), jnp.float32)
    ea_aug = ea_aug.at[:E, :32].set(edge_attr).at[:E, 32].set(1.0)
    dstp = jnp.concatenate([dst, jnp.zeros((E_PAD - E,), jnp.int32)])
    p_attr = sc_scatter_add(dstp, ea_aug, E_PAD, GP_N, 10240, E)
    loop_attr = tc_loopattr(p_attr)

    ea_all = jnp.concatenate(
        [edge_attr, loop_attr, jnp.zeros((M_PAD - E - N, 32), jnp.float32)])
    ze1 = tc_matmul(ea_all, _padw(We1), jnp.zeros((8, P), jnp.float32), 1024)
    ze2 = tc_matmul(ea_all, _padw(We2), jnp.zeros((8, P), jnp.float32), 1024)

    att1p = jnp.zeros((8, P), jnp.float32).at[0, :100].set(att1)
    att2p = jnp.zeros((8, P), jnp.float32).at[0, :100].set(att2)

    h1 = _gat_layer(x, s1d, d1d, ze1, Wl1, bl1, Wr1, br1, att1p,
                    _padb(bias1, False), relu=True)
    h2 = _gat_layer(h1, s1d, d1d, ze2, Wl2, bl2, Wr2, br2, att2p,
                    _padb(bias2, False), relu=False)

    h2p = jnp.zeros((NP_PAD, P), jnp.float32).at[:N].set(h2)
    bat = jnp.concatenate(
        [batch.astype(jnp.int32), jnp.zeros((NP_PAD - N,), jnp.int32)])
    p_pool = sc_scatter_add(bat, h2p, NP_PAD, GP_B, 512, N)

    wlinp = jnp.zeros((P, 8), jnp.float32).at[:100, 0].set(Wlin[:, 0])
    blinp = jnp.broadcast_to(blin.reshape(1, 1), (8, 8)).astype(jnp.float32)
    out = tc_head(p_pool, wlinp, blinp)
    return out[:, :1]
